# Initial kernel scaffold; baseline (speedup 1.0000x reference)
import jax
import jax.numpy as jnp
from jax.experimental import pallas as pl


def kernel(x, edge_index, edge_weights, W1, W2, W3, b1, b2, b3, lin_W, lin_b):
    raise NotImplementedError("write your pallas kernel here")


# speedup vs baseline: 9.8761x; 9.8761x over previous
import dataclasses
import functools

import jax
import jax.numpy as jnp
from jax import lax
from jax.experimental import pallas as pl
from jax.experimental.pallas import tpu as pltpu
from jax.experimental.pallas import tpu_sc as plsc

N = 10000
E = 320000
F_IN = 128
H = 20
OUT = 20
C = 40

NC = 2
NS = 16
NW = NC * NS
CH = 128
ROWS_PER_W = 80
EP = NW * ROWS_PER_W * CH
ER = EP // CH
NP = 10240
NPT = NP // NS
HP = 32
BLK = 1024


def _spmm_body(srcr, dstr, wr, hs, out, src_v, dst_v, w_sm, rows_v, zbuf,
               acc_sh):
    c = lax.axis_index("c")
    s = lax.axis_index("s")
    wid = c * NS + s

    @pl.loop(0, NPT)
    def _(i):
        zbuf[i, pl.ds(0, 16)] = jnp.zeros((16,), jnp.float32)
        zbuf[i, pl.ds(16, 16)] = jnp.zeros((16,), jnp.float32)

    pltpu.sync_copy(zbuf, acc_sh.at[pl.ds(s * NPT, NPT)])
    plsc.subcore_barrier()

    @pl.loop(0, ROWS_PER_W)
    def _(j):
        r = wid * ROWS_PER_W + j
        pltpu.sync_copy(srcr.at[r], src_v)
        pltpu.sync_copy(dstr.at[r], dst_v)
        pltpu.sync_copy(wr.at[r], w_sm)
        pltpu.sync_copy(hs.at[src_v], rows_v)

        @pl.loop(0, CH)
        def _(k):
            wk = plsc.load_gather(w_sm, [jnp.full((16,), k, jnp.int32)])
            rows_v[k, pl.ds(0, 16)] = rows_v[k, pl.ds(0, 16)] * wk
            rows_v[k, pl.ds(16, 16)] = rows_v[k, pl.ds(16, 16)] * wk

        pltpu.sync_copy(rows_v, acc_sh.at[dst_v], add=True)

    plsc.subcore_barrier()
    pltpu.sync_copy(acc_sh.at[pl.ds(s * NPT, NPT)],
                    out.at[c].at[pl.ds(s * NPT, NPT)])


def _deg_body(dstr, wr, out, dst_v, w_v, zbuf, deg_sh):
    c = lax.axis_index("c")
    s = lax.axis_index("s")
    wid = c * NS + s

    @pl.loop(0, NPT, step=16)
    def _(i):
        zbuf[pl.ds(i, 16)] = jnp.zeros((16,), jnp.float32)

    pltpu.sync_copy(zbuf, deg_sh.at[pl.ds(s * NPT, NPT)])
    plsc.subcore_barrier()

    @pl.loop(0, ROWS_PER_W)
    def _(j):
        r = wid * ROWS_PER_W + j
        pltpu.sync_copy(dstr.at[r], dst_v)
        pltpu.sync_copy(wr.at[r], w_v)
        pltpu.sync_copy(w_v, deg_sh.at[dst_v], add=True)

    plsc.subcore_barrier()
    pltpu.sync_copy(deg_sh.at[pl.ds(s * NPT, NPT)],
                    out.at[c].at[pl.ds(s * NPT, NPT)])


def _make_sc_kernels():
    mesh = plsc.VectorSubcoreMesh(core_axis_name="c", subcore_axis_name="s")
    cp = pltpu.CompilerParams()
    if "needs_layout_passes" in pltpu.CompilerParams.__dataclass_fields__:
        cp = dataclasses.replace(cp, needs_layout_passes=False)
    if "use_tc_tiling_on_sc" in pltpu.CompilerParams.__dataclass_fields__:
        cp = dataclasses.replace(cp, use_tc_tiling_on_sc=False)
    spmm = functools.partial(
        pl.kernel, mesh=mesh, compiler_params=cp,
        out_type=jax.ShapeDtypeStruct((NC, NP, HP), jnp.float32),
        scratch_types=[
            pltpu.VMEM((CH,), jnp.int32),
            pltpu.VMEM((CH,), jnp.int32),
            pltpu.VMEM((CH,), jnp.float32),
            pltpu.VMEM((CH, HP), jnp.float32),
            pltpu.VMEM((NPT, HP), jnp.float32),
            pltpu.VMEM_SHARED((NP, HP), jnp.float32),
        ])(_spmm_body)
    deg = functools.partial(
        pl.kernel, mesh=mesh,
        out_type=jax.ShapeDtypeStruct((NC, NP), jnp.float32),
        scratch_types=[
            pltpu.VMEM((CH,), jnp.int32),
            pltpu.VMEM((CH,), jnp.float32),
            pltpu.VMEM((NPT,), jnp.float32),
            pltpu.VMEM_SHARED((NP,), jnp.float32),
        ])(_deg_body)
    return spmm, deg


def _mm_body(x_ref, w_ref, o_ref):
    o_ref[...] = jnp.dot(x_ref[...], w_ref[...],
                         preferred_element_type=jnp.float32)


def _hs1_body(d0_ref, d1_ref, h1_ref, hs_ref, dv_ref):
    deg = d0_ref[...] + d1_ref[...] + 1.0
    dinv = jnp.where(deg > 0, lax.rsqrt(jnp.maximum(deg, 1e-12)), 0.0)
    dv = jnp.broadcast_to(dinv, (BLK, HP))
    dv_ref[...] = dv
    hs_ref[...] = h1_ref[...] * dv


def _layer_body(a0_ref, a1_ref, hs_ref, dv_ref, w_ref, b_ref,
                x_ref, hsn_ref, *, double_relu):
    conv = dv_ref[...] * (a0_ref[...] + a1_ref[...] + hs_ref[...])
    if double_relu:
        xl = jax.nn.relu(jax.nn.relu(conv) + b_ref[...])
    else:
        xl = jax.nn.relu(conv + b_ref[...])
    x_ref[...] = xl
    hsn_ref[...] = dv_ref[...] * jnp.dot(xl, w_ref[...],
                                         preferred_element_type=jnp.float32)


def _final_body(a0_ref, a1_ref, hs_ref, dv_ref, b_ref, x1_ref, x2_ref,
                lw_ref, lb_ref, o_ref):
    conv = dv_ref[...] * (a0_ref[...] + a1_ref[...] + hs_ref[...])
    x3 = jax.nn.relu(conv + b_ref[...])
    hcat = jnp.concatenate(
        [x1_ref[...][:, :H], x2_ref[...][:, :H], x3[:, :OUT],
         jnp.zeros((BLK, 4), jnp.float32)], axis=1)
    logits = jnp.dot(hcat, lw_ref[...],
                     preferred_element_type=jnp.float32) + lb_ref[...]
    m = jnp.max(logits, axis=1, keepdims=True)
    lse = jnp.log(jnp.sum(jnp.exp(logits - m), axis=1, keepdims=True)) + m
    o_ref[...] = logits - lse


def _row_spec(w):
    return pl.BlockSpec((BLK, w), lambda i: (i, 0))


def _full_spec(shape):
    return pl.BlockSpec(shape, lambda i: (0, 0))


_GRID = (NP // BLK,)


def _tc_matmul(x_p, w):
    return pl.pallas_call(
        _mm_body, grid=_GRID,
        in_specs=[_row_spec(F_IN), _full_spec((F_IN, HP))],
        out_specs=_row_spec(HP),
        out_shape=jax.ShapeDtypeStruct((NP, HP), jnp.float32))(x_p, w)


def _tc_hs1(d0, d1, h1):
    return pl.pallas_call(
        _hs1_body, grid=_GRID,
        in_specs=[_row_spec(1), _row_spec(1), _row_spec(HP)],
        out_specs=[_row_spec(HP), _row_spec(HP)],
        out_shape=[jax.ShapeDtypeStruct((NP, HP), jnp.float32),
                   jax.ShapeDtypeStruct((NP, HP), jnp.float32)])(d0, d1, h1)


def _tc_layer(a0, a1, hs, dv, w, b, double_relu):
    body = functools.partial(_layer_body, double_relu=double_relu)
    return pl.pallas_call(
        body, grid=_GRID,
        in_specs=[_row_spec(HP), _row_spec(HP), _row_spec(HP), _row_spec(HP),
                  _full_spec((HP, HP)), _full_spec((1, HP))],
        out_specs=[_row_spec(HP), _row_spec(HP)],
        out_shape=[jax.ShapeDtypeStruct((NP, HP), jnp.float32),
                   jax.ShapeDtypeStruct((NP, HP), jnp.float32)])(
                       a0, a1, hs, dv, w, b)


def _tc_final(a0, a1, hs, dv, b, x1, x2, lw, lb):
    return pl.pallas_call(
        _final_body, grid=_GRID,
        in_specs=[_row_spec(HP), _row_spec(HP), _row_spec(HP), _row_spec(HP),
                  _full_spec((1, HP)), _row_spec(HP), _row_spec(HP),
                  _full_spec((64, C)), _full_spec((1, C))],
        out_specs=_row_spec(C),
        out_shape=jax.ShapeDtypeStruct((NP, C), jnp.float32))(
            a0, a1, hs, dv, b, x1, x2, lw, lb)


def kernel(x, edge_index, edge_weights, W1, W2, W3, b1, b2, b3, lin_W, lin_b):
    f32 = jnp.float32
    pad_e = EP - E
    src_r = jnp.concatenate(
        [edge_index[0], jnp.zeros((pad_e,), jnp.int32)]).reshape(ER, CH)
    dst_r = jnp.concatenate(
        [edge_index[1], jnp.zeros((pad_e,), jnp.int32)]).reshape(ER, CH)
    w_r = jnp.concatenate(
        [edge_weights, jnp.zeros((pad_e,), f32)]).reshape(ER, CH)
    x_p = jnp.pad(x, ((0, NP - N), (0, 0)))
    w1t = jnp.pad(W1.T, ((0, 0), (0, HP - H)))
    w2t = jnp.pad(W2.T, ((0, HP - H), (0, HP - H)))
    w3t = jnp.pad(W3.T, ((0, HP - H), (0, HP - OUT)))
    b1r = jnp.pad(b1, (0, HP - H)).reshape(1, HP)
    b2r = jnp.pad(b2, (0, HP - H)).reshape(1, HP)
    b3r = jnp.pad(b3, (0, HP - OUT)).reshape(1, HP)
    lwt = jnp.pad(lin_W.T, ((0, 64 - (2 * H + OUT)), (0, 0)))
    lbr = lin_b.reshape(1, C)

    spmm, deg_kernel = _make_sc_kernels()

    degp = deg_kernel(dst_r, w_r)
    h1 = _tc_matmul(x_p, w1t)
    d0 = degp[0].reshape(NP, 1)
    d1 = degp[1].reshape(NP, 1)
    hs1, dv = _tc_hs1(d0, d1, h1)
    acc1 = spmm(src_r, dst_r, w_r, hs1)
    x1, hs2 = _tc_layer(acc1[0], acc1[1], hs1, dv, w2t, b1r, False)
    acc2 = spmm(src_r, dst_r, w_r, hs2)
    x2, hs3 = _tc_layer(acc2[0], acc2[1], hs2, dv, w3t, b2r, True)
    acc3 = spmm(src_r, dst_r, w_r, hs3)
    outp = _tc_final(acc3[0], acc3[1], hs3, dv, b3r, x1, x2, lwt, lbr)
    return outp[:N]
